# baseline (speedup 1.0000x reference)
import functools

import jax
import jax.numpy as jnp
from jax import lax
from jax.experimental import pallas as pl
from jax.experimental.pallas import tpu as pltpu
from jax.experimental.pallas import tpu_sc as plsc

N = 512
H = 128
CUTOFF = 0.3
PPP = 2.0
INV_TWO_SIGMA_SQ = 2.0
REG = 0.01

_NC = 2
_NS = 16
_L = 16
_CB = 128
_NCB = N // _CB
_RG = (_NC * _NS) // _NCB
_RPW = N // _RG
_CHUNKS = _CB // _L


def _sc_partial_colsums(dist):
    mesh = plsc.VectorSubcoreMesh(core_axis_name="c", subcore_axis_name="s")

    @functools.partial(
        pl.kernel,
        out_type=jax.ShapeDtypeStruct((_RG * N,), jnp.float32),
        mesh=mesh,
        scratch_types=[
            pltpu.VMEM((_RPW, _CB), jnp.float32),
            pltpu.VMEM((_CB,), jnp.float32),
        ],
    )
    def k(dist_hbm, part_hbm, buf, acc_v):
        wid = lax.axis_index("s") * _NC + lax.axis_index("c")
        cb = wid // _RG
        rg = wid % _RG
        pltpu.sync_copy(
            dist_hbm.at[pl.ds(rg * _RPW, _RPW), pl.ds(cb * _CB, _CB)], buf)

        def sens_chunk(i, c):
            d = buf[i, pl.ds(c * _L, _L)]
            r = 1.0 / d - 1.0
            s = jnp.exp(r * r * -INV_TWO_SIGMA_SQ)
            return jnp.where(d < CUTOFF, s, 0.0)

        def body(i, accs):
            return tuple(accs[c] + sens_chunk(i, c) for c in range(_CHUNKS))

        accs = lax.fori_loop(
            0, _RPW, body,
            tuple(jnp.zeros((_L,), jnp.float32) for _ in range(_CHUNKS)),
        )
        for c in range(_CHUNKS):
            acc_v[pl.ds(c * _L, _L)] = accs[c]
        pltpu.sync_copy(acc_v, part_hbm.at[pl.ds(rg * N + cb * _CB, _CB)])

    return k(dist)


def _tc_body(geom_ref, w1_ref, b1_ref, w2_ref, b2_ref, wp_ref,
             ppp_ref, loss_ref):
    x = jnp.dot(geom_ref[...], w1_ref[...],
                preferred_element_type=jnp.float32) + b1_ref[...]
    z_on = jnp.maximum(x, 0.0) + jnp.log1p(jnp.exp(-jnp.abs(x)))
    colsum = jnp.sum(z_on, axis=0, keepdims=True)
    w = jnp.sum(wp_ref[...], axis=0, keepdims=True)
    u = jnp.dot(w, z_on, preferred_element_type=jnp.float32)
    corr = (colsum
            + jnp.dot(u, w2_ref[...], preferred_element_type=jnp.float32)
            + jnp.sum(w) * b2_ref[...])
    ppp_ref[...] = PPP + corr
    loss_ref[...] = REG * jnp.sqrt(jnp.sum(corr * corr, keepdims=True))


def _tc_combine(geom, W1, B1, W2, B2, w_partials):
    ppp, loss = pl.pallas_call(
        _tc_body,
        out_shape=(
            jax.ShapeDtypeStruct((1, H), jnp.float32),
            jax.ShapeDtypeStruct((1, 1), jnp.float32),
        ),
    )(geom, W1, B1.reshape(1, H), W2, B2.reshape(1, H),
      w_partials.reshape(_RG, N))
    return ppp.reshape(H), loss.reshape(())


def kernel(geom_array, dist_matrix, W1, B1, W2, B2):
    w_partials = _sc_partial_colsums(dist_matrix)
    ppp, loss = _tc_combine(geom_array, W1, B1, W2, B2, w_partials)
    return ppp, loss
